# baseline (speedup 1.0000x reference)
import functools

import jax
import jax.numpy as jnp
import numpy as np
from jax.experimental import pallas as pl
from jax.experimental.pallas import tpu as pltpu

EPS = 1e-5


def _square_distance(a, b):
    d = jnp.sum(a * a, -1)[:, None] + jnp.sum(b * b, -1)[None, :] - 2.0 * (a @ b.T)
    return jnp.maximum(d, 0.0)


def _fps(p, npoint):
    n = p.shape[0]

    def body(i, state):
        idx, dists, far = state
        idx = idx.at[i].set(far)
        d = jnp.sum((p - p[far]) ** 2, -1)
        dists = jnp.minimum(dists, d)
        return idx, dists, jnp.argmax(dists).astype(jnp.int32)

    idx0 = jnp.zeros((npoint,), jnp.int32)
    d0 = jnp.full((n,), 1e10, jnp.float32)
    idx, _, _ = jax.lax.fori_loop(0, npoint, body, (idx0, d0, jnp.int32(0)))
    return idx


def _knn_idx(q, r, k):
    d = _square_distance(q, r)
    _, idx = jax.lax.top_k(-d, k)
    return idx


def _gather(a, idx):
    return jax.vmap(lambda x, i: x[i])(a, idx)


def _bn(x, g, b):
    axes = tuple(range(x.ndim - 1))
    mu = jnp.mean(x, axes, keepdims=True)
    var = jnp.var(x, axes, keepdims=True)
    return g * (x - mu) / jnp.sqrt(var + EPS) + b


def _mlp(h, layers):
    for (W, g, b) in layers:
        h = jax.nn.relu(_bn(h @ W, g, b))
    return h


def _sa(p, x, npoint, nsample, layers):
    fidx = jax.vmap(lambda pp: _fps(pp, npoint))(p)
    new_p = _gather(p, fidx)
    nidx = jax.vmap(lambda q, r: _knn_idx(q, r, nsample))(new_p, p)
    gp = _gather(p, nidx) - new_p[:, :, None, :]
    gx = _gather(x, nidx)
    h = _mlp(jnp.concatenate([gp, gx], -1), layers)
    return new_p, jnp.max(h, 2)


def _fp(p1, x1, p2, x2, layers):
    d = jax.vmap(_square_distance)(p1, p2)
    negv, idx = jax.lax.top_k(-d, 3)
    dist = jnp.maximum(-negv, 0.0)
    w = 1.0 / (dist + 1e-8)
    w = w / jnp.sum(w, -1, keepdims=True)
    xi = jnp.sum(w[..., None] * _gather(x2, idx), 2)
    h = xi if x1 is None else jnp.concatenate([x1, xi], -1)
    return _mlp(h, layers)


def _seg_head_body(x_ref, w1_ref, b1_ref, g_ref, bt_ref, w2_ref, b2_ref,
                   stat_ref, o_ref):
    h = x_ref[...] @ w1_ref[...] + b1_ref[...][None, :]
    mu = stat_ref[0, :]
    var = stat_ref[1, :]
    h = g_ref[...][None, :] * (h - mu[None, :]) / jnp.sqrt(var[None, :] + EPS) \
        + bt_ref[...][None, :]
    o_ref[...] = h @ w2_ref[...] + b2_ref[...][None, :]


def _seg_head(x, s):
    h = x @ s['W1'] + s['b1']
    mu = jnp.mean(h, 0)
    var = jnp.var(h, 0)
    stat = jnp.stack([mu, var], 0)
    n = x.shape[0]
    blk = 2048
    grid = (n // blk,)
    return pl.pallas_call(
        _seg_head_body,
        grid=grid,
        in_specs=[
            pl.BlockSpec((blk, 128), lambda i: (i, 0)),
            pl.BlockSpec((128, 128), lambda i: (0, 0)),
            pl.BlockSpec((128,), lambda i: (0,)),
            pl.BlockSpec((128,), lambda i: (0,)),
            pl.BlockSpec((128,), lambda i: (0,)),
            pl.BlockSpec((128, 8), lambda i: (0, 0)),
            pl.BlockSpec((8,), lambda i: (0,)),
            pl.BlockSpec((2, 128), lambda i: (0, 0)),
        ],
        out_specs=pl.BlockSpec((blk, 8), lambda i: (i, 0)),
        out_shape=jax.ShapeDtypeStruct((n, 8), jnp.float32),
    )(x, s['W1'], s['b1'], s['g'], s['bt'],
      jnp.pad(s['W2'], ((0, 0), (0, 2))), jnp.pad(s['b2'], (0, 2)), stat)[:, :6]


def kernel(coord, feat, params, offset):
    B = int(offset.shape[0])
    n = coord.shape[0] // B
    p0 = coord.reshape(B, n, 3)
    x0 = feat.reshape(B, n, -1)
    p1, x1 = _sa(p0, x0, 1024, 32, params['sa1'])
    p2, x2 = _sa(p1, x1, 256, 32, params['sa2'])
    p3, x3 = _sa(p2, x2, 64, 32, params['sa3'])
    p4, x4 = _sa(p3, x3, 16, 32, params['sa4'])
    x3n = _fp(p3, x3, p4, x4, params['fp4'])
    x2n = _fp(p2, x2, p3, x3n, params['fp3'])
    x1n = _fp(p1, x1, p2, x2n, params['fp2'])
    x0n = _fp(p0, None, p1, x1n, params['fp1'])
    out = _seg_head(x0n.reshape(B * n, -1), params['seg'])
    return out


# speedup vs baseline: 3.4756x; 3.4756x over previous
import functools

import jax
import jax.numpy as jnp
import numpy as np
from jax.experimental import pallas as pl
from jax.experimental.pallas import tpu as pltpu

EPS = 1e-5


def _fps_body(npoint, n, B, px_ref, py_ref, pz_ref,
              idx_out, npx_out, npy_out, npz_out):
    px = px_ref[...]
    py = py_ref[...]
    pz = pz_ref[...]
    iota = jax.lax.broadcasted_iota(jnp.int32, (B, n), 1)
    col = jax.lax.broadcasted_iota(jnp.int32, (B, npoint), 1)

    def it(i, c):
        dists, far, fx, fy, fz = c
        selcol = col == i
        idx_out[...] = jnp.where(selcol, far, idx_out[...])
        npx_out[...] = jnp.where(selcol, fx, npx_out[...])
        npy_out[...] = jnp.where(selcol, fy, npy_out[...])
        npz_out[...] = jnp.where(selcol, fz, npz_out[...])
        d = ((px - fx) * (px - fx) + (pz - fz) * (pz - fz)) + (py - fy) * (py - fy)
        dists = jnp.minimum(dists, d)
        m = jnp.max(dists, axis=1, keepdims=True)
        eq = dists == m
        nidx = jnp.min(jnp.where(eq, iota, n), axis=1, keepdims=True)
        oh = iota == nidx
        nfx = jnp.sum(jnp.where(oh, px, 0.0), axis=1, keepdims=True)
        nfy = jnp.sum(jnp.where(oh, py, 0.0), axis=1, keepdims=True)
        nfz = jnp.sum(jnp.where(oh, pz, 0.0), axis=1, keepdims=True)
        return (dists, nidx, nfx, nfy, nfz)

    d0 = jnp.full((B, n), 1e10, jnp.float32)
    far0 = jnp.zeros((B, 1), jnp.int32)
    c0 = (d0, far0, px[:, 0:1], py[:, 0:1], pz[:, 0:1])
    jax.lax.fori_loop(0, npoint, it, c0)


def _fps_pallas(p, npoint):
    B, n, _ = p.shape
    px, py, pz = p[:, :, 0], p[:, :, 1], p[:, :, 2]
    outs = pl.pallas_call(
        functools.partial(_fps_body, npoint, n, B),
        out_shape=[
            jax.ShapeDtypeStruct((B, npoint), jnp.int32),
            jax.ShapeDtypeStruct((B, npoint), jnp.float32),
            jax.ShapeDtypeStruct((B, npoint), jnp.float32),
            jax.ShapeDtypeStruct((B, npoint), jnp.float32),
        ],
    )(px, py, pz)
    idx, npx, npy, npz = outs
    new_p = jnp.stack([npx, npy, npz], axis=-1)
    return idx, new_p


def _knn_body(k, N, QB, want_w, q_ref, r_ref, rx_ref, ry_ref, rz_ref,
              idx_out, *maybe_w):
    qv = q_ref[0]
    rv = r_ref[0]
    qx, qy, qz = qv[:, 0:1], qv[:, 1:2], qv[:, 2:3]
    A = (qx * qx + qz * qz) + qy * qy
    rx, ry, rz = rx_ref[0], ry_ref[0], rz_ref[0]
    Bv = (rx * rx + rz * rz) + ry * ry
    C = jax.lax.dot_general(qv, rv, (((1,), (1,)), ((), ())),
                            preferred_element_type=jnp.float32)
    D = jnp.maximum(A + Bv - 2.0 * C, 0.0)
    iota = jax.lax.broadcasted_iota(jnp.int32, (QB, N), 1)
    ms = []
    for j in range(k):
        m = jnp.min(D, axis=1, keepdims=True)
        eq = D == m
        idx = jnp.min(jnp.where(eq, iota, N), axis=1, keepdims=True)
        idx_out[0, :, j:j + 1] = idx
        if want_w:
            ms.append(m)
        if j + 1 < k:
            D = jnp.where(iota == idx, 1e30, D)
    if want_w:
        w_out = maybe_w[0]
        ws = [1.0 / (m + 1e-8) for m in ms]
        s = ws[0]
        for t in ws[1:]:
            s = s + t
        for j in range(k):
            w_out[0, :, j:j + 1] = ws[j] / s


def _topk_pallas(q, r, k, qb, want_w=False):
    B, Q, _ = q.shape
    N = r.shape[1]
    grid = (B, Q // qb)
    out_shape = [jax.ShapeDtypeStruct((B, Q, k), jnp.int32)]
    out_specs = [pl.BlockSpec((1, qb, k), lambda b, i: (b, i, 0))]
    if want_w:
        out_shape.append(jax.ShapeDtypeStruct((B, Q, k), jnp.float32))
        out_specs.append(pl.BlockSpec((1, qb, k), lambda b, i: (b, i, 0)))
    outs = pl.pallas_call(
        functools.partial(_knn_body, k, N, qb, want_w),
        grid=grid,
        in_specs=[
            pl.BlockSpec((1, qb, 3), lambda b, i: (b, i, 0)),
            pl.BlockSpec((1, N, 3), lambda b, i: (b, 0, 0)),
            pl.BlockSpec((1, 1, N), lambda b, i: (b, 0, 0)),
            pl.BlockSpec((1, 1, N), lambda b, i: (b, 0, 0)),
            pl.BlockSpec((1, 1, N), lambda b, i: (b, 0, 0)),
        ],
        out_specs=out_specs,
        out_shape=out_shape,
    )(q, r, r[:, None, :, 0], r[:, None, :, 1], r[:, None, :, 2])
    return outs if want_w else (outs[0],)


def _gather(a, idx):
    return jax.vmap(lambda x, i: x[i])(a, idx)


def _bn(x, g, b):
    axes = tuple(range(x.ndim - 1))
    mu = jnp.mean(x, axes, keepdims=True)
    var = jnp.var(x, axes, keepdims=True)
    return g * (x - mu) / jnp.sqrt(var + EPS) + b


def _mlp(h, layers):
    for (W, g, b) in layers:
        h = jax.nn.relu(_bn(h @ W, g, b))
    return h


def _sa(p, x, npoint, nsample, layers, qb):
    _, new_p = _fps_pallas(p, npoint)
    (nidx,) = _topk_pallas(new_p, p, nsample, qb)
    gp = _gather(p, nidx) - new_p[:, :, None, :]
    gx = _gather(x, nidx)
    h = _mlp(jnp.concatenate([gp, gx], -1), layers)
    return new_p, jnp.max(h, 2)


def _fp(p1, x1, p2, x2, layers, qb):
    idx, w = _topk_pallas(p1, p2, 3, qb, want_w=True)
    xi = jnp.sum(w[..., None] * _gather(x2, idx), 2)
    h = xi if x1 is None else jnp.concatenate([x1, xi], -1)
    return _mlp(h, layers)


def _seg_head_body(x_ref, w1_ref, b1_ref, g_ref, bt_ref, w2_ref, b2_ref,
                   stat_ref, o_ref):
    h = x_ref[...] @ w1_ref[...] + b1_ref[...][None, :]
    mu = stat_ref[0, :]
    var = stat_ref[1, :]
    h = g_ref[...][None, :] * (h - mu[None, :]) / jnp.sqrt(var[None, :] + EPS) \
        + bt_ref[...][None, :]
    o_ref[...] = h @ w2_ref[...] + b2_ref[...][None, :]


def _seg_head(x, s):
    h = x @ s['W1'] + s['b1']
    mu = jnp.mean(h, 0)
    var = jnp.var(h, 0)
    stat = jnp.stack([mu, var], 0)
    n = x.shape[0]
    blk = 2048
    grid = (n // blk,)
    return pl.pallas_call(
        _seg_head_body,
        grid=grid,
        in_specs=[
            pl.BlockSpec((blk, 128), lambda i: (i, 0)),
            pl.BlockSpec((128, 128), lambda i: (0, 0)),
            pl.BlockSpec((128,), lambda i: (0,)),
            pl.BlockSpec((128,), lambda i: (0,)),
            pl.BlockSpec((128,), lambda i: (0,)),
            pl.BlockSpec((128, 8), lambda i: (0, 0)),
            pl.BlockSpec((8,), lambda i: (0,)),
            pl.BlockSpec((2, 128), lambda i: (0, 0)),
        ],
        out_specs=pl.BlockSpec((blk, 8), lambda i: (i, 0)),
        out_shape=jax.ShapeDtypeStruct((n, 8), jnp.float32),
    )(x, s['W1'], s['b1'], s['g'], s['bt'],
      jnp.pad(s['W2'], ((0, 0), (0, 2))), jnp.pad(s['b2'], (0, 2)), stat)[:, :6]


def kernel(coord, feat, params, offset):
    B = int(offset.shape[0])
    n = coord.shape[0] // B
    p0 = coord.reshape(B, n, 3)
    x0 = feat.reshape(B, n, -1)
    p1, x1 = _sa(p0, x0, 1024, 32, params['sa1'], 256)
    p2, x2 = _sa(p1, x1, 256, 32, params['sa2'], 256)
    p3, x3 = _sa(p2, x2, 64, 32, params['sa3'], 64)
    p4, x4 = _sa(p3, x3, 16, 32, params['sa4'], 16)
    x3n = _fp(p3, x3, p4, x4, params['fp4'], 64)
    x2n = _fp(p2, x2, p3, x3n, params['fp3'], 256)
    x1n = _fp(p1, x1, p2, x2n, params['fp2'], 512)
    x0n = _fp(p0, None, p1, x1n, params['fp1'], 512)
    out = _seg_head(x0n.reshape(B * n, -1), params['seg'])
    return out


# speedup vs baseline: 10.9150x; 3.1405x over previous
import functools

import jax
import jax.numpy as jnp
import numpy as np
from jax import lax
from jax.experimental import pallas as pl
from jax.experimental.pallas import tpu as pltpu
from jax.experimental.pallas import tpu_sc as plsc

EPS = 1e-5


def _sc_gather(table, idx, chunk=None):
    V, D = table.shape
    M = idx.shape[0]
    NW = 32
    assert M % (8 * NW) == 0 and D % 16 == 0
    b_per_w = M // NW
    if chunk is None:
        chunk = b_per_w
    nch = b_per_w // chunk
    mesh = plsc.VectorSubcoreMesh(core_axis_name="c", subcore_axis_name="s")

    @functools.partial(
        pl.kernel, mesh=mesh,
        out_type=jax.ShapeDtypeStruct((M, D), jnp.float32),
        scratch_types=[
            pltpu.VMEM((b_per_w,), jnp.int32),
            pltpu.VMEM((chunk, D), jnp.float32),
            pltpu.SemaphoreType.DMA,
        ],
    )
    def k(table_hbm, idx_hbm, out_hbm, idx_v, rows_v, sem):
        wid = lax.axis_index("s") * 2 + lax.axis_index("c")
        base = wid * b_per_w
        pltpu.sync_copy(idx_hbm.at[pl.ds(base, b_per_w)], idx_v)
        for c in range(nch):
            pltpu.async_copy(table_hbm.at[idx_v.at[pl.ds(c * chunk, chunk)]],
                             rows_v, sem).wait()
            pltpu.sync_copy(rows_v, out_hbm.at[pl.ds(base + c * chunk, chunk)])

    return k(table, idx)


def _fps_body(npoint, n, B, px_ref, py_ref, pz_ref,
              idx_out, npx_out, npy_out, npz_out):
    px = px_ref[...]
    py = py_ref[...]
    pz = pz_ref[...]
    iota = jax.lax.broadcasted_iota(jnp.int32, (B, n), 1)
    col = jax.lax.broadcasted_iota(jnp.int32, (B, npoint), 1)

    def it(i, c):
        dists, far, fx, fy, fz = c
        selcol = col == i
        idx_out[...] = jnp.where(selcol, far, idx_out[...])
        npx_out[...] = jnp.where(selcol, fx, npx_out[...])
        npy_out[...] = jnp.where(selcol, fy, npy_out[...])
        npz_out[...] = jnp.where(selcol, fz, npz_out[...])
        d = ((px - fx) * (px - fx) + (pz - fz) * (pz - fz)) + (py - fy) * (py - fy)
        dists = jnp.minimum(dists, d)
        m = jnp.max(dists, axis=1, keepdims=True)
        eq = dists == m
        nidx = jnp.min(jnp.where(eq, iota, n), axis=1, keepdims=True)
        oh = iota == nidx
        nfx = jnp.sum(jnp.where(oh, px, 0.0), axis=1, keepdims=True)
        nfy = jnp.sum(jnp.where(oh, py, 0.0), axis=1, keepdims=True)
        nfz = jnp.sum(jnp.where(oh, pz, 0.0), axis=1, keepdims=True)
        return (dists, nidx, nfx, nfy, nfz)

    d0 = jnp.full((B, n), 1e10, jnp.float32)
    far0 = jnp.zeros((B, 1), jnp.int32)
    c0 = (d0, far0, px[:, 0:1], py[:, 0:1], pz[:, 0:1])
    jax.lax.fori_loop(0, npoint, it, c0)


def _fps_pallas(p, npoint):
    B, n, _ = p.shape
    px, py, pz = p[:, :, 0], p[:, :, 1], p[:, :, 2]
    outs = pl.pallas_call(
        functools.partial(_fps_body, npoint, n, B),
        out_shape=[
            jax.ShapeDtypeStruct((B, npoint), jnp.int32),
            jax.ShapeDtypeStruct((B, npoint), jnp.float32),
            jax.ShapeDtypeStruct((B, npoint), jnp.float32),
            jax.ShapeDtypeStruct((B, npoint), jnp.float32),
        ],
    )(px, py, pz)
    idx, npx, npy, npz = outs
    new_p = jnp.stack([npx, npy, npz], axis=-1)
    return idx, new_p


def _knn_body(k, N, QB, want_w, q_ref, r_ref, rx_ref, ry_ref, rz_ref,
              idx_out, *maybe_w):
    qv = q_ref[0]
    rv = r_ref[0]
    qx, qy, qz = qv[:, 0:1], qv[:, 1:2], qv[:, 2:3]
    A = (qx * qx + qz * qz) + qy * qy
    rx, ry, rz = rx_ref[0], ry_ref[0], rz_ref[0]
    Bv = (rx * rx + rz * rz) + ry * ry
    C = jax.lax.dot_general(qv, rv, (((1,), (1,)), ((), ())),
                            preferred_element_type=jnp.float32)
    D = jnp.maximum(A + Bv - 2.0 * C, 0.0)
    iota = jax.lax.broadcasted_iota(jnp.int32, (QB, N), 1)
    ms = []
    for j in range(k):
        m = jnp.min(D, axis=1, keepdims=True)
        eq = D == m
        idx = jnp.min(jnp.where(eq, iota, N), axis=1, keepdims=True)
        idx_out[0, :, j:j + 1] = idx
        if want_w:
            ms.append(m)
        if j + 1 < k:
            D = jnp.where(iota == idx, 1e30, D)
    if want_w:
        w_out = maybe_w[0]
        ws = [1.0 / (m + 1e-8) for m in ms]
        s = ws[0]
        for t in ws[1:]:
            s = s + t
        for j in range(k):
            w_out[0, :, j:j + 1] = ws[j] / s


def _topk_pallas(q, r, k, qb, want_w=False):
    B, Q, _ = q.shape
    N = r.shape[1]
    grid = (B, Q // qb)
    out_shape = [jax.ShapeDtypeStruct((B, Q, k), jnp.int32)]
    out_specs = [pl.BlockSpec((1, qb, k), lambda b, i: (b, i, 0))]
    if want_w:
        out_shape.append(jax.ShapeDtypeStruct((B, Q, k), jnp.float32))
        out_specs.append(pl.BlockSpec((1, qb, k), lambda b, i: (b, i, 0)))
    outs = pl.pallas_call(
        functools.partial(_knn_body, k, N, qb, want_w),
        grid=grid,
        in_specs=[
            pl.BlockSpec((1, qb, 3), lambda b, i: (b, i, 0)),
            pl.BlockSpec((1, N, 3), lambda b, i: (b, 0, 0)),
            pl.BlockSpec((1, 1, N), lambda b, i: (b, 0, 0)),
            pl.BlockSpec((1, 1, N), lambda b, i: (b, 0, 0)),
            pl.BlockSpec((1, 1, N), lambda b, i: (b, 0, 0)),
        ],
        out_specs=out_specs,
        out_shape=out_shape,
    )(q, r, r[:, None, :, 0], r[:, None, :, 1], r[:, None, :, 2])
    return outs if want_w else (outs[0],)


def _gather(a, idx):
    return jax.vmap(lambda x, i: x[i])(a, idx)


def _bn(x, g, b):
    axes = tuple(range(x.ndim - 1))
    mu = jnp.mean(x, axes, keepdims=True)
    var = jnp.var(x, axes, keepdims=True)
    return g * (x - mu) / jnp.sqrt(var + EPS) + b


def _mlp(h, layers):
    for (W, g, b) in layers:
        h = jax.nn.relu(_bn(h @ W, g, b))
    return h


def _sa(p, x, npoint, nsample, layers, qb, chunk=None):
    B, N, _ = p.shape
    C = x.shape[-1]
    _, new_p = _fps_pallas(p, npoint)
    (nidx,) = _topk_pallas(new_p, p, nsample, qb)
    Cp = ((3 + C + 127) // 128) * 128
    table = jnp.concatenate(
        [p, x, jnp.zeros((B, N, Cp - 3 - C), jnp.float32)], -1).reshape(B * N, Cp)
    gidx = (nidx + (jnp.arange(B, dtype=jnp.int32) * N)[:, None, None]).reshape(-1)
    rows = _sc_gather(table, gidx, chunk=chunk).reshape(B, npoint, nsample, Cp)
    gp = rows[..., :3] - new_p[:, :, None, :]
    gx = rows[..., 3:3 + C]
    h = _mlp(jnp.concatenate([gp, gx], -1), layers)
    return new_p, jnp.max(h, 2)


def _fp(p1, x1, p2, x2, layers, qb, chunk=None):
    B, Q, _ = p1.shape
    N2, C2 = p2.shape[1], x2.shape[-1]
    idx, w = _topk_pallas(p1, p2, 3, qb, want_w=True)
    gidx = (idx + (jnp.arange(B, dtype=jnp.int32) * N2)[:, None, None]).reshape(-1)
    rows = _sc_gather(x2.reshape(B * N2, C2), gidx, chunk=chunk)
    xi = jnp.sum(w[..., None] * rows.reshape(B, Q, 3, C2), 2)
    h = xi if x1 is None else jnp.concatenate([x1, xi], -1)
    return _mlp(h, layers)


def _seg_head_body(x_ref, w1_ref, b1_ref, g_ref, bt_ref, w2_ref, b2_ref,
                   stat_ref, o_ref):
    h = x_ref[...] @ w1_ref[...] + b1_ref[...][None, :]
    mu = stat_ref[0, :]
    var = stat_ref[1, :]
    h = g_ref[...][None, :] * (h - mu[None, :]) / jnp.sqrt(var[None, :] + EPS) \
        + bt_ref[...][None, :]
    o_ref[...] = h @ w2_ref[...] + b2_ref[...][None, :]


def _seg_head(x, s):
    h = x @ s['W1'] + s['b1']
    mu = jnp.mean(h, 0)
    var = jnp.var(h, 0)
    stat = jnp.stack([mu, var], 0)
    n = x.shape[0]
    blk = 2048
    grid = (n // blk,)
    return pl.pallas_call(
        _seg_head_body,
        grid=grid,
        in_specs=[
            pl.BlockSpec((blk, 128), lambda i: (i, 0)),
            pl.BlockSpec((128, 128), lambda i: (0, 0)),
            pl.BlockSpec((128,), lambda i: (0,)),
            pl.BlockSpec((128,), lambda i: (0,)),
            pl.BlockSpec((128,), lambda i: (0,)),
            pl.BlockSpec((128, 8), lambda i: (0, 0)),
            pl.BlockSpec((8,), lambda i: (0,)),
            pl.BlockSpec((2, 128), lambda i: (0, 0)),
        ],
        out_specs=pl.BlockSpec((blk, 8), lambda i: (i, 0)),
        out_shape=jax.ShapeDtypeStruct((n, 8), jnp.float32),
    )(x, s['W1'], s['b1'], s['g'], s['bt'],
      jnp.pad(s['W2'], ((0, 0), (0, 2))), jnp.pad(s['b2'], (0, 2)), stat)[:, :6]


def kernel(coord, feat, params, offset):
    B = int(offset.shape[0])
    n = coord.shape[0] // B
    p0 = coord.reshape(B, n, 3)
    x0 = feat.reshape(B, n, -1)
    p1, x1 = _sa(p0, x0, 1024, 32, params['sa1'], 256, chunk=512)
    p2, x2 = _sa(p1, x1, 256, 32, params['sa2'], 256, chunk=512)
    p3, x3 = _sa(p2, x2, 64, 32, params['sa3'], 64)
    p4, x4 = _sa(p3, x3, 16, 32, params['sa4'], 16)
    x3n = _fp(p3, x3, p4, x4, params['fp4'], 64)
    x2n = _fp(p2, x2, p3, x3n, params['fp3'], 256)
    x1n = _fp(p1, x1, p2, x2n, params['fp2'], 512)
    x0n = _fp(p0, None, p1, x1n, params['fp1'], 512, chunk=512)
    out = _seg_head(x0n.reshape(B * n, -1), params['seg'])
    return out


# speedup vs baseline: 10.9287x; 1.0013x over previous
import functools

import jax
import jax.numpy as jnp
import numpy as np
from jax import lax
from jax.experimental import pallas as pl
from jax.experimental.pallas import tpu as pltpu
from jax.experimental.pallas import tpu_sc as plsc

EPS = 1e-5


def _sc_gather(table, idx, chunk=None):
    V, D = table.shape
    M = idx.shape[0]
    NW = 32
    assert M % (8 * NW) == 0 and D % 16 == 0
    b_per_w = M // NW
    if chunk is None:
        chunk = b_per_w
    nch = b_per_w // chunk
    mesh = plsc.VectorSubcoreMesh(core_axis_name="c", subcore_axis_name="s")

    @functools.partial(
        pl.kernel, mesh=mesh,
        out_type=jax.ShapeDtypeStruct((M, D), jnp.float32),
        scratch_types=[
            pltpu.VMEM((b_per_w,), jnp.int32),
            pltpu.VMEM((chunk, D), jnp.float32),
            pltpu.SemaphoreType.DMA,
        ],
    )
    def k(table_hbm, idx_hbm, out_hbm, idx_v, rows_v, sem):
        wid = lax.axis_index("s") * 2 + lax.axis_index("c")
        base = wid * b_per_w
        pltpu.sync_copy(idx_hbm.at[pl.ds(base, b_per_w)], idx_v)
        for c in range(nch):
            pltpu.async_copy(table_hbm.at[idx_v.at[pl.ds(c * chunk, chunk)]],
                             rows_v, sem).wait()
            pltpu.sync_copy(rows_v, out_hbm.at[pl.ds(base + c * chunk, chunk)])

    return k(table, idx)


def _fps_body(npoint, n, B, px_ref, py_ref, pz_ref,
              idx_out, npx_out, npy_out, npz_out):
    px = px_ref[...]
    py = py_ref[...]
    pz = pz_ref[...]
    iota = jax.lax.broadcasted_iota(jnp.int32, (B, n), 1)
    col = jax.lax.broadcasted_iota(jnp.int32, (B, npoint), 1)

    def it(i, c):
        dists, far, fx, fy, fz = c
        selcol = col == i
        idx_out[...] = jnp.where(selcol, far, idx_out[...])
        npx_out[...] = jnp.where(selcol, fx, npx_out[...])
        npy_out[...] = jnp.where(selcol, fy, npy_out[...])
        npz_out[...] = jnp.where(selcol, fz, npz_out[...])
        d = ((px - fx) * (px - fx) + (py - fy) * (py - fy)) + (pz - fz) * (pz - fz)
        dists = jnp.minimum(dists, d)
        m = jnp.max(dists, axis=1, keepdims=True)
        eq = dists == m
        nidx = jnp.min(jnp.where(eq, iota, n), axis=1, keepdims=True)
        oh = iota == nidx
        nfx = jnp.sum(jnp.where(oh, px, 0.0), axis=1, keepdims=True)
        nfy = jnp.sum(jnp.where(oh, py, 0.0), axis=1, keepdims=True)
        nfz = jnp.sum(jnp.where(oh, pz, 0.0), axis=1, keepdims=True)
        return (dists, nidx, nfx, nfy, nfz)

    d0 = jnp.full((B, n), 1e10, jnp.float32)
    far0 = jnp.zeros((B, 1), jnp.int32)
    c0 = (d0, far0, px[:, 0:1], py[:, 0:1], pz[:, 0:1])
    jax.lax.fori_loop(0, npoint, it, c0)


def _fps_pallas(p, npoint):
    B, n, _ = p.shape
    px, py, pz = p[:, :, 0], p[:, :, 1], p[:, :, 2]
    outs = pl.pallas_call(
        functools.partial(_fps_body, npoint, n, B),
        out_shape=[
            jax.ShapeDtypeStruct((B, npoint), jnp.int32),
            jax.ShapeDtypeStruct((B, npoint), jnp.float32),
            jax.ShapeDtypeStruct((B, npoint), jnp.float32),
            jax.ShapeDtypeStruct((B, npoint), jnp.float32),
        ],
    )(px, py, pz)
    idx, npx, npy, npz = outs
    new_p = jnp.stack([npx, npy, npz], axis=-1)
    return idx, new_p


def _knn_body(k, N, QB, want_w, q_ref, r_ref, rx_ref, ry_ref, rz_ref,
              idx_out, *maybe_w):
    qv = q_ref[0]
    rv = r_ref[0]
    qx, qy, qz = qv[:, 0:1], qv[:, 1:2], qv[:, 2:3]
    A = (qx * qx + qz * qz) + qy * qy
    rx, ry, rz = rx_ref[0], ry_ref[0], rz_ref[0]
    Bv = (rx * rx + rz * rz) + ry * ry
    C = jax.lax.dot_general(qv, rv, (((1,), (1,)), ((), ())),
                            preferred_element_type=jnp.float32)
    D = jnp.maximum(A + Bv - 2.0 * C, 0.0)
    iota = jax.lax.broadcasted_iota(jnp.int32, (QB, N), 1)
    ms = []
    for j in range(k):
        m = jnp.min(D, axis=1, keepdims=True)
        eq = D == m
        idx = jnp.min(jnp.where(eq, iota, N), axis=1, keepdims=True)
        idx_out[0, :, j:j + 1] = idx
        if want_w:
            ms.append(m)
        if j + 1 < k:
            D = jnp.where(iota == idx, 1e30, D)
    if want_w:
        w_out = maybe_w[0]
        ws = [1.0 / (m + 1e-8) for m in ms]
        s = ws[0]
        for t in ws[1:]:
            s = s + t
        for j in range(k):
            w_out[0, :, j:j + 1] = ws[j] / s


def _topk_pallas(q, r, k, qb, want_w=False):
    B, Q, _ = q.shape
    N = r.shape[1]
    grid = (B, Q // qb)
    out_shape = [jax.ShapeDtypeStruct((B, Q, k), jnp.int32)]
    out_specs = [pl.BlockSpec((1, qb, k), lambda b, i: (b, i, 0))]
    if want_w:
        out_shape.append(jax.ShapeDtypeStruct((B, Q, k), jnp.float32))
        out_specs.append(pl.BlockSpec((1, qb, k), lambda b, i: (b, i, 0)))
    outs = pl.pallas_call(
        functools.partial(_knn_body, k, N, qb, want_w),
        grid=grid,
        in_specs=[
            pl.BlockSpec((1, qb, 3), lambda b, i: (b, i, 0)),
            pl.BlockSpec((1, N, 3), lambda b, i: (b, 0, 0)),
            pl.BlockSpec((1, 1, N), lambda b, i: (b, 0, 0)),
            pl.BlockSpec((1, 1, N), lambda b, i: (b, 0, 0)),
            pl.BlockSpec((1, 1, N), lambda b, i: (b, 0, 0)),
        ],
        out_specs=out_specs,
        out_shape=out_shape,
    )(q, r, r[:, None, :, 0], r[:, None, :, 1], r[:, None, :, 2])
    return outs if want_w else (outs[0],)


def _gather(a, idx):
    return jax.vmap(lambda x, i: x[i])(a, idx)


def _bn(x, g, b):
    axes = tuple(range(x.ndim - 1))
    mu = jnp.mean(x, axes, keepdims=True)
    var = jnp.var(x, axes, keepdims=True)
    return g * (x - mu) / jnp.sqrt(var + EPS) + b


def _mlp(h, layers):
    for (W, g, b) in layers:
        h = jax.nn.relu(_bn(h @ W, g, b))
    return h


def _sa(p, x, npoint, nsample, layers, qb, chunk=None):
    B, N, _ = p.shape
    C = x.shape[-1]
    _, new_p = _fps_pallas(p, npoint)
    (nidx,) = _topk_pallas(new_p, p, nsample, qb)
    Cp = ((3 + C + 127) // 128) * 128
    table = jnp.concatenate(
        [p, x, jnp.zeros((B, N, Cp - 3 - C), jnp.float32)], -1).reshape(B * N, Cp)
    gidx = (nidx + (jnp.arange(B, dtype=jnp.int32) * N)[:, None, None]).reshape(-1)
    rows = _sc_gather(table, gidx, chunk=chunk).reshape(B, npoint, nsample, Cp)
    gp = rows[..., :3] - new_p[:, :, None, :]
    gx = rows[..., 3:3 + C]
    h = _mlp(jnp.concatenate([gp, gx], -1), layers)
    return new_p, jnp.max(h, 2)


def _fp(p1, x1, p2, x2, layers, qb, chunk=None):
    B, Q, _ = p1.shape
    N2, C2 = p2.shape[1], x2.shape[-1]
    idx, w = _topk_pallas(p1, p2, 3, qb, want_w=True)
    gidx = (idx + (jnp.arange(B, dtype=jnp.int32) * N2)[:, None, None]).reshape(-1)
    rows = _sc_gather(x2.reshape(B * N2, C2), gidx, chunk=chunk)
    xi = jnp.sum(w[..., None] * rows.reshape(B, Q, 3, C2), 2)
    h = xi if x1 is None else jnp.concatenate([x1, xi], -1)
    return _mlp(h, layers)


def _seg_head_body(x_ref, w1_ref, b1_ref, g_ref, bt_ref, w2_ref, b2_ref,
                   stat_ref, o_ref):
    h = x_ref[...] @ w1_ref[...] + b1_ref[...][None, :]
    mu = stat_ref[0, :]
    var = stat_ref[1, :]
    h = g_ref[...][None, :] * (h - mu[None, :]) / jnp.sqrt(var[None, :] + EPS) \
        + bt_ref[...][None, :]
    o_ref[...] = h @ w2_ref[...] + b2_ref[...][None, :]


def _seg_head(x, s):
    h = x @ s['W1'] + s['b1']
    mu = jnp.mean(h, 0)
    var = jnp.var(h, 0)
    stat = jnp.stack([mu, var], 0)
    n = x.shape[0]
    blk = 2048
    grid = (n // blk,)
    return pl.pallas_call(
        _seg_head_body,
        grid=grid,
        in_specs=[
            pl.BlockSpec((blk, 128), lambda i: (i, 0)),
            pl.BlockSpec((128, 128), lambda i: (0, 0)),
            pl.BlockSpec((128,), lambda i: (0,)),
            pl.BlockSpec((128,), lambda i: (0,)),
            pl.BlockSpec((128,), lambda i: (0,)),
            pl.BlockSpec((128, 8), lambda i: (0, 0)),
            pl.BlockSpec((8,), lambda i: (0,)),
            pl.BlockSpec((2, 128), lambda i: (0, 0)),
        ],
        out_specs=pl.BlockSpec((blk, 8), lambda i: (i, 0)),
        out_shape=jax.ShapeDtypeStruct((n, 8), jnp.float32),
    )(x, s['W1'], s['b1'], s['g'], s['bt'],
      jnp.pad(s['W2'], ((0, 0), (0, 2))), jnp.pad(s['b2'], (0, 2)), stat)[:, :6]


def kernel(coord, feat, params, offset):
    B = int(offset.shape[0])
    n = coord.shape[0] // B
    p0 = coord.reshape(B, n, 3)
    x0 = feat.reshape(B, n, -1)
    p1, x1 = _sa(p0, x0, 1024, 32, params['sa1'], 256, chunk=512)
    p2, x2 = _sa(p1, x1, 256, 32, params['sa2'], 256, chunk=512)
    p3, x3 = _sa(p2, x2, 64, 32, params['sa3'], 64)
    p4, x4 = _sa(p3, x3, 16, 32, params['sa4'], 16)
    x3n = _fp(p3, x3, p4, x4, params['fp4'], 64)
    x2n = _fp(p2, x2, p3, x3n, params['fp3'], 256)
    x1n = _fp(p1, x1, p2, x2n, params['fp2'], 512)
    x0n = _fp(p0, None, p1, x1n, params['fp1'], 512, chunk=512)
    out = _seg_head(x0n.reshape(B * n, -1), params['seg'])
    return out
